# baseline (speedup 1.0000x reference)
import functools

import jax
import jax.numpy as jnp
from jax import lax
from jax.experimental import pallas as pl
from jax.experimental.pallas import tpu as pltpu
from jax.experimental.pallas import tpu_sc as plsc

B = 16384
D = 64
L = 16
NC = 2
NS = 16
NW = NC * NS
BPW = B // NW
C = 256
NCHUNK = BPW // C
NGRP = C // L
DG = D // L


def _sc_body(h_hbm, t_hbm, r_hbm, ent1_hbm, ent2_hbm, rel1_hbm, rel2_hbm,
             res_hbm,
             hc, tc, rc, e1h, e2h, e1t, e2t, r1c, r2c, pbuf, resc, sem):
    wid = lax.axis_index("s") * NC + lax.axis_index("c")
    row_ids = lax.iota(jnp.int32, L)

    for chunk in range(NCHUNK):
        base = wid * BPW + chunk * C
        pltpu.sync_copy(h_hbm.at[pl.ds(base, C)], hc)
        pltpu.sync_copy(t_hbm.at[pl.ds(base, C)], tc)
        pltpu.sync_copy(r_hbm.at[pl.ds(base, C)], rc)

        cps = [
            pltpu.async_copy(ent1_hbm.at[hc], e1h, sem),
            pltpu.async_copy(ent2_hbm.at[hc], e2h, sem),
            pltpu.async_copy(ent1_hbm.at[tc], e1t, sem),
            pltpu.async_copy(ent2_hbm.at[tc], e2t, sem),
            pltpu.async_copy(rel1_hbm.at[rc], r1c, sem),
            pltpu.async_copy(rel2_hbm.at[rc], r2c, sem),
        ]
        for cp in cps:
            cp.wait()

        def grp_body(g, _):
            res_v = jnp.zeros((L,), jnp.float32)
            for e in range(L):
                eb = g * L + e
                acc = jnp.zeros((L,), jnp.float32)
                for dg in range(DG):
                    sl = pl.ds(dg * L, L)
                    a1 = e1h[eb, sl]
                    a2 = e2h[eb, sl]
                    b1 = e1t[eb, sl]
                    b2 = e2t[eb, sl]
                    q1 = r1c[eb, sl]
                    q2 = r2c[eb, sl]
                    acc = acc + q1 * (a1 * b1 + a2 * b2) + q2 * (a1 * b2 - a2 * b1)
                s = jnp.sum(acc)
                res_v = jnp.where(row_ids == e, s, res_v)
            resc[pl.ds(g * L, L)] = res_v
            return 0

        lax.fori_loop(0, NGRP, grp_body, 0)
        pltpu.sync_copy(resc, res_hbm.at[pl.ds(base, C)])


def _make_sc_kernel():
    mesh = plsc.VectorSubcoreMesh(core_axis_name="c", subcore_axis_name="s")
    return pl.kernel(
        _sc_body,
        out_type=jax.ShapeDtypeStruct((B,), jnp.float32),
        mesh=mesh,
        compiler_params=pltpu.CompilerParams(
            needs_layout_passes=False, use_tc_tiling_on_sc=False),
        scratch_types=[
            pltpu.VMEM((C,), jnp.int32),
            pltpu.VMEM((C,), jnp.int32),
            pltpu.VMEM((C,), jnp.int32),
            pltpu.VMEM((C, D), jnp.float32),
            pltpu.VMEM((C, D), jnp.float32),
            pltpu.VMEM((C, D), jnp.float32),
            pltpu.VMEM((C, D), jnp.float32),
            pltpu.VMEM((C, D), jnp.float32),
            pltpu.VMEM((C, D), jnp.float32),
            pltpu.VMEM((L * L,), jnp.float32),
            pltpu.VMEM((C,), jnp.float32),
            pltpu.SemaphoreType.DMA,
        ],
    )


def _loss_body(res_ref, y_ref, out_ref):
    x = -y_ref[...] * res_ref[...]
    out_ref[0, 0] = jnp.mean(jax.nn.softplus(x))


@jax.jit
def kernel(h, t, r, y, ent1, ent2, rel1, rel2):
    h = h.astype(jnp.int32)
    t = t.astype(jnp.int32)
    r = r.astype(jnp.int32)
    res = _make_sc_kernel()(h, t, r, ent1, ent2, rel1, rel2)
    loss = pl.pallas_call(
        _loss_body,
        out_shape=jax.ShapeDtypeStruct((1, 1), jnp.float32),
        out_specs=pl.BlockSpec(memory_space=pltpu.SMEM),
    )(res.reshape(128, 128), y.reshape(128, 128))
    return loss[0, 0]


# speedup vs baseline: 1.5284x; 1.5284x over previous
import functools

import jax
import jax.numpy as jnp
from jax import lax
from jax.experimental import pallas as pl
from jax.experimental.pallas import tpu as pltpu
from jax.experimental.pallas import tpu_sc as plsc

B = 16384
D = 64
L = 16
NC = 2
NS = 16
NW = NC * NS
BPW = B // NW
C = 128
NCHUNK = BPW // C
NGRP = C // L
DG = D // L


def _sc_body(h_hbm, t_hbm, r_hbm, ent1_hbm, ent2_hbm, rel1_hbm, rel2_hbm,
             res_hbm,
             hv, tv, rv, e1h, e2h, e1t, e2t, r1c, r2c, pbuf, resc,
             sem):
    wid = lax.axis_index("s") * NC + lax.axis_index("c")
    row_ids = lax.iota(jnp.int32, L)

    for chunk in range(NCHUNK):
        base = wid * BPW + chunk * C
        pltpu.sync_copy(h_hbm.at[pl.ds(base, C)], hv.at[pl.ds(0, C)])
        pltpu.sync_copy(t_hbm.at[pl.ds(base, C)], tv.at[pl.ds(0, C)])
        pltpu.sync_copy(r_hbm.at[pl.ds(base, C)], rv.at[pl.ds(0, C)])

        def fetch_body(e, _):
            kh = hv[pl.ds(e, L)][0]
            kt = tv[pl.ds(e, L)][0]
            kr = rv[pl.ds(e, L)][0]
            pltpu.async_copy(ent1_hbm.at[kh], e1h.at[e], sem)
            pltpu.async_copy(ent2_hbm.at[kh], e2h.at[e], sem)
            pltpu.async_copy(ent1_hbm.at[kt], e1t.at[e], sem)
            pltpu.async_copy(ent2_hbm.at[kt], e2t.at[e], sem)
            pltpu.async_copy(rel1_hbm.at[kr], r1c.at[e], sem)
            pltpu.async_copy(rel2_hbm.at[kr], r2c.at[e], sem)
            return 0

        lax.fori_loop(0, C, fetch_body, 0)
        for buf in (e1h, e2h, e1t, e2t, r1c, r2c):
            pltpu.make_async_copy(ent1_hbm.at[pl.ds(0, C)], buf, sem).wait()

        def grp_body(g, _):
            res_v = jnp.zeros((L,), jnp.float32)
            for e in range(L):
                eb = g * L + e
                acc = jnp.zeros((L,), jnp.float32)
                for dg in range(DG):
                    sl = pl.ds(dg * L, L)
                    a1 = e1h[eb, sl]
                    a2 = e2h[eb, sl]
                    b1 = e1t[eb, sl]
                    b2 = e2t[eb, sl]
                    q1 = r1c[eb, sl]
                    q2 = r2c[eb, sl]
                    acc = acc + q1 * (a1 * b1 + a2 * b2) + q2 * (a1 * b2 - a2 * b1)
                s = jnp.sum(acc)
                res_v = jnp.where(row_ids == e, s, res_v)
            resc[pl.ds(g * L, L)] = res_v
            return 0

        lax.fori_loop(0, NGRP, grp_body, 0)
        pltpu.sync_copy(resc, res_hbm.at[pl.ds(base, C)])


def _make_sc_kernel():
    mesh = plsc.VectorSubcoreMesh(core_axis_name="c", subcore_axis_name="s")
    return pl.kernel(
        _sc_body,
        out_type=jax.ShapeDtypeStruct((B,), jnp.float32),
        mesh=mesh,
        compiler_params=pltpu.CompilerParams(
            needs_layout_passes=False, use_tc_tiling_on_sc=True),
        scratch_types=[
            pltpu.VMEM((C + L,), jnp.int32),
            pltpu.VMEM((C + L,), jnp.int32),
            pltpu.VMEM((C + L,), jnp.int32),
            pltpu.VMEM((C, D), jnp.float32),
            pltpu.VMEM((C, D), jnp.float32),
            pltpu.VMEM((C, D), jnp.float32),
            pltpu.VMEM((C, D), jnp.float32),
            pltpu.VMEM((C, D), jnp.float32),
            pltpu.VMEM((C, D), jnp.float32),
            pltpu.VMEM((L * L,), jnp.float32),
            pltpu.VMEM((C,), jnp.float32),
            pltpu.SemaphoreType.DMA,
        ],
    )


def _loss_body(res_ref, y_ref, out_ref):
    x = -y_ref[...] * res_ref[...]
    out_ref[0, 0] = jnp.mean(jax.nn.softplus(x))


@jax.jit
def kernel(h, t, r, y, ent1, ent2, rel1, rel2):
    h = h.astype(jnp.int32)
    t = t.astype(jnp.int32)
    r = r.astype(jnp.int32)
    res = _make_sc_kernel()(h, t, r, ent1, ent2, rel1, rel2)
    loss = pl.pallas_call(
        _loss_body,
        out_shape=jax.ShapeDtypeStruct((1, 1), jnp.float32),
        out_specs=pl.BlockSpec(memory_space=pltpu.SMEM),
    )(res.reshape(128, 128), y.reshape(128, 128))
    return loss[0, 0]
